# baseline (speedup 1.0000x reference)
import functools

import jax
import jax.numpy as jnp
from jax import lax
from jax.experimental import pallas as pl
from jax.experimental.pallas import tpu as pltpu
from jax.experimental.pallas import tpu_sc as plsc

N = 10000
H = 128
E = 320000
EDGE_DIM = 16
XW = 16

NC = 2
NS = 16
NW = NC * NS

GC = 80
GEPW = E // NW
GITERS = GEPW // GC


GPAIRS = (GITERS - 1) // 2


@functools.partial(
    pl.kernel,
    out_type=[
        jax.ShapeDtypeStruct((E, H), jnp.float32),
        jax.ShapeDtypeStruct((E, H), jnp.float32),
        jax.ShapeDtypeStruct((E, XW), jnp.float32),
    ],
    mesh=plsc.VectorSubcoreMesh(core_axis_name="c", subcore_axis_name="s"),
    compiler_params=pltpu.CompilerParams(needs_layout_passes=False),
    scratch_types=[
        pltpu.VMEM((GEPW,), jnp.int32),
        pltpu.VMEM((GEPW,), jnp.int32),
        pltpu.VMEM((GC, H), jnp.float32),
        pltpu.VMEM((GC, H), jnp.float32),
        pltpu.VMEM((GC, H), jnp.float32),
        pltpu.VMEM((GC, H), jnp.float32),
        pltpu.VMEM((GC, XW), jnp.float32),
        pltpu.VMEM((GC, XW), jnp.float32),
        pltpu.VMEM((N,), jnp.float32),
        pltpu.VMEM((N,), jnp.float32),
        pltpu.VMEM((N,), jnp.float32),
        pltpu.SemaphoreType.DMA,
        pltpu.SemaphoreType.DMA,
    ],
)
def _sc_gather(p_hbm, q_hbm, x0_hbm, x1_hbm, x2_hbm, row_hbm, col_hbm,
               hr_out, hc_out, cdrad_out,
               idxr_v, idxc_v, hr0_v, hc0_v, hr1_v, hc1_v, cdrad0_v, cdrad1_v,
               x0_v, x1_v, x2_v, sem0, sem1):
    c = lax.axis_index("c")
    s = lax.axis_index("s")
    base = (s * NC + c) * GEPW
    pltpu.sync_copy(row_hbm.at[pl.ds(base, GEPW)], idxr_v)
    pltpu.sync_copy(col_hbm.at[pl.ds(base, GEPW)], idxc_v)
    pltpu.sync_copy(x0_hbm, x0_v)
    pltpu.sync_copy(x1_hbm, x1_v)
    pltpu.sync_copy(x2_hbm, x2_v)

    def _mk(i, hr_v, hc_v, sem):
        off = i * GC
        c1 = pltpu.make_async_copy(p_hbm.at[idxr_v.at[pl.ds(off, GC)]], hr_v, sem)
        c2 = pltpu.make_async_copy(q_hbm.at[idxc_v.at[pl.ds(off, GC)]], hc_v, sem)
        return c1, c2

    def _start(i, hr_v, hc_v, sem):
        c1, c2 = _mk(i, hr_v, hc_v, sem)
        c1.start()
        c2.start()

    def _wait(i, hr_v, hc_v, sem):
        c1, c2 = _mk(i, hr_v, hc_v, sem)
        c1.wait()
        c2.wait()

    def _geo(i, cdrad_v):
        off = i * GC
        for j in range(GC // 16):
            ir = idxr_v[pl.ds(off + j * 16, 16)]
            ic = idxc_v[pl.ds(off + j * 16, 16)]
            cd0 = plsc.load_gather(x0_v, [ir]) - plsc.load_gather(x0_v, [ic])
            cd1 = plsc.load_gather(x1_v, [ir]) - plsc.load_gather(x1_v, [ic])
            cd2 = plsc.load_gather(x2_v, [ir]) - plsc.load_gather(x2_v, [ic])
            rad = cd0 * cd0 + cd1 * cd1 + cd2 * cd2
            rows = jax.lax.iota(jnp.int32, 16) + j * 16
            for comp, val in ((0, cd0), (1, cd1), (2, cd2), (3, rad)):
                cols = jnp.full((16,), comp, jnp.int32)
                plsc.store_scatter(cdrad_v, [rows, cols], val)

    def _outs(i, hr_v, hc_v, cdrad_v):
        off = base + i * GC
        pltpu.sync_copy(hr_v, hr_out.at[pl.ds(off, GC)])
        pltpu.sync_copy(hc_v, hc_out.at[pl.ds(off, GC)])
        pltpu.sync_copy(cdrad_v, cdrad_out.at[pl.ds(off, GC)])

    _start(0, hr0_v, hc0_v, sem0)

    def body(p, carry):
        a = 2 * p
        b = a + 1
        _start(b, hr1_v, hc1_v, sem1)
        _geo(a, cdrad0_v)
        _wait(a, hr0_v, hc0_v, sem0)
        _outs(a, hr0_v, hc0_v, cdrad0_v)
        _start(b + 1, hr0_v, hc0_v, sem0)
        _geo(b, cdrad1_v)
        _wait(b, hr1_v, hc1_v, sem1)
        _outs(b, hr1_v, hc1_v, cdrad1_v)
        return carry

    lax.fori_loop(0, GPAIRS, body, 0)
    last = GITERS - 1
    _geo(last, cdrad0_v)
    _wait(last, hr0_v, hc0_v, sem0)
    _outs(last, hr0_v, hc0_v, cdrad0_v)


SC2 = 80
SITERS = E // NW // SC2
SEPC = E // NC
SEPW = E // NW
NPT = 640
NPAD = NPT * NS
RCH = 32


def _make_sc_scatter(width, tc_tiling):
    @functools.partial(
        pl.kernel,
        out_type=jax.ShapeDtypeStruct((NC * NPAD, width), jnp.float32),
        mesh=plsc.VectorSubcoreMesh(core_axis_name="c", subcore_axis_name="s"),
        compiler_params=pltpu.CompilerParams(
            needs_layout_passes=False, use_tc_tiling_on_sc=tc_tiling),
        scratch_types=[
            pltpu.VMEM((SC2,), jnp.int32),
            pltpu.VMEM((SC2,), jnp.int32),
            pltpu.VMEM((SC2, width), jnp.float32),
            pltpu.VMEM((SC2, width), jnp.float32),
            pltpu.VMEM((RCH, width), jnp.float32),
            pltpu.VMEM_SHARED((NPAD, width), jnp.float32),
            pltpu.SemaphoreType.DMA,
            pltpu.SemaphoreType.DMA,
        ],
    )
    def _scatter(v_hbm, row_hbm, z_hbm, part,
                 idx0_v, idx1_v, v0_v, v1_v, rb_v, acc, sem0, sem1):
        c = lax.axis_index("c")
        s = lax.axis_index("s")
        for k in range(NPT // RCH):
            r0 = s * NPT + k * RCH
            pltpu.sync_copy(z_hbm.at[pl.ds(r0, RCH)], rb_v)
            pltpu.sync_copy(rb_v, acc.at[pl.ds(r0, RCH)])
        plsc.subcore_barrier()

        base = c * SEPC + s * SEPW

        def _mk2(i, idx_v, v_v, sem):
            off = base + i * SC2
            return (pltpu.make_async_copy(row_hbm.at[pl.ds(off, SC2)], idx_v, sem),
                    pltpu.make_async_copy(v_hbm.at[pl.ds(off, SC2)], v_v, sem))

        def _start2(i, idx_v, v_v, sem):
            for cp in _mk2(i, idx_v, v_v, sem):
                cp.start()

        def _wait2(i, idx_v, v_v, sem):
            for cp in _mk2(i, idx_v, v_v, sem):
                cp.wait()

        _start2(0, idx0_v, v0_v, sem0)

        def body(p, carry):
            a = 2 * p
            b = a + 1
            _start2(b, idx1_v, v1_v, sem1)
            _wait2(a, idx0_v, v0_v, sem0)
            pltpu.sync_copy(v0_v, acc.at[idx0_v], add=True)
            _start2(b + 1, idx0_v, v0_v, sem0)
            _wait2(b, idx1_v, v1_v, sem1)
            pltpu.sync_copy(v1_v, acc.at[idx1_v], add=True)
            return carry

        lax.fori_loop(0, (SITERS - 1) // 2, body, 0)
        last = SITERS - 1
        _wait2(last, idx0_v, v0_v, sem0)
        pltpu.sync_copy(v0_v, acc.at[idx0_v], add=True)
        plsc.subcore_barrier()
        for k in range(NPT // RCH):
            r0 = s * NPT + k * RCH
            pltpu.sync_copy(acc.at[pl.ds(r0, RCH)], rb_v)
            pltpu.sync_copy(rb_v, part.at[pl.ds(c * NPAD + r0, RCH)])

    return _scatter


_sc_scatter_m = _make_sc_scatter(H, True)
_sc_scatter_x = _make_sc_scatter(XW, False)


EB = 2000
ENB = E // EB


def _silu(v):
    return v * jax.nn.sigmoid(v)


def _prenode_body(h, w1r, w1c, p_out, q_out):
    p_out[...] = jnp.dot(h[...], w1r[...], preferred_element_type=jnp.float32)
    q_out[...] = jnp.dot(h[...], w1c[...], preferred_element_type=jnp.float32)


def _prenode(h, w1r, w1c):
    wspec = lambda shp: pl.BlockSpec(shp, lambda i: (0, 0))
    return pl.pallas_call(
        _prenode_body,
        grid=(NNB,),
        in_specs=[pl.BlockSpec((NB, H), lambda i: (i, 0)), wspec((H, H)), wspec((H, H))],
        out_specs=[pl.BlockSpec((NB, H), lambda i: (i, 0)),
                   pl.BlockSpec((NB, H), lambda i: (i, 0))],
        out_shape=[jax.ShapeDtypeStruct((N, H), jnp.float32),
                   jax.ShapeDtypeStruct((N, H), jnp.float32)],
    )(h, w1r, w1c)


def _edge_body(pr, qc, cdrad, ea, w1rad, w1a, b1e, w2e, b2e,
               wc1, bc1, wc2t, m_out, cu_out):
    lane = lax.broadcasted_iota(jnp.int32, (1, XW), 1)
    cd = jnp.where(lane < 3, cdrad[...], 0.0)
    radial = cdrad[:, 3:4]
    z1 = (pr[...] + qc[...]
          + jnp.dot(ea[...], w1a[...], preferred_element_type=jnp.float32)
          + radial * w1rad[...]
          + b1e[...])
    t1 = _silu(z1)
    m = _silu(jnp.dot(t1, w2e[...], preferred_element_type=jnp.float32) + b2e[...])
    t3 = _silu(jnp.dot(m, wc1[...], preferred_element_type=jnp.float32) + bc1[...])
    cw = jnp.sum(t3 * wc2t[...], axis=1, keepdims=True)
    m_out[...] = m
    cu_out[...] = cd * cw


def _edge_mlp(pr, qc, cdrad, ea, w1rad, w1a, b1e, w2e, b2e, wc1, bc1, wc2t):
    wspec = lambda shp: pl.BlockSpec(shp, lambda i: (0, 0))
    return pl.pallas_call(
        _edge_body,
        grid=(ENB,),
        in_specs=[
            pl.BlockSpec((EB, H), lambda i: (i, 0)),
            pl.BlockSpec((EB, H), lambda i: (i, 0)),
            pl.BlockSpec((EB, XW), lambda i: (i, 0)),
            pl.BlockSpec((EB, EDGE_DIM), lambda i: (i, 0)),
            wspec((1, H)), wspec((EDGE_DIM, H)),
            wspec((1, H)), wspec((H, H)), wspec((1, H)),
            wspec((H, H)), wspec((1, H)), wspec((1, H)),
        ],
        out_specs=[
            pl.BlockSpec((EB, H), lambda i: (i, 0)),
            pl.BlockSpec((EB, XW), lambda i: (i, 0)),
        ],
        out_shape=[
            jax.ShapeDtypeStruct((E, H), jnp.float32),
            jax.ShapeDtypeStruct((E, XW), jnp.float32),
        ],
    )(pr, qc, cdrad, ea, w1rad, w1a, b1e, w2e, b2e, wc1, bc1, wc2t)


NB = 1000
NNB = N // NB


def _node_body(h, m0, m1, x16, xp0, xp1, wn1a, wn1b, bn1, wn2, bn2, hn_out, xn_out):
    mi = m0[...] + m1[...]
    z = _silu(jnp.dot(h[...], wn1a[...], preferred_element_type=jnp.float32)
              + jnp.dot(mi, wn1b[...], preferred_element_type=jnp.float32)
              + bn1[...])
    hn_out[...] = h[...] + jnp.dot(z, wn2[...], preferred_element_type=jnp.float32) + bn2[...]
    xn_out[...] = x16[...] + xp0[...] + xp1[...]


def _node_mlp(h, m0, m1, x16, xp0, xp1, wn1a, wn1b, bn1, wn2, bn2):
    wspec = lambda shp: pl.BlockSpec(shp, lambda i: (0, 0))
    return pl.pallas_call(
        _node_body,
        grid=(NNB,),
        in_specs=[
            pl.BlockSpec((NB, H), lambda i: (i, 0)),
            pl.BlockSpec((NB, H), lambda i: (i, 0)),
            pl.BlockSpec((NB, H), lambda i: (i, 0)),
            pl.BlockSpec((NB, XW), lambda i: (i, 0)),
            pl.BlockSpec((NB, XW), lambda i: (i, 0)),
            pl.BlockSpec((NB, XW), lambda i: (i, 0)),
            wspec((H, H)), wspec((H, H)), wspec((1, H)), wspec((H, H)), wspec((1, H)),
        ],
        out_specs=[
            pl.BlockSpec((NB, H), lambda i: (i, 0)),
            pl.BlockSpec((NB, XW), lambda i: (i, 0)),
        ],
        out_shape=[
            jax.ShapeDtypeStruct((N, H), jnp.float32),
            jax.ShapeDtypeStruct((N, XW), jnp.float32),
        ],
    )(h, m0, m1, x16, xp0, xp1, wn1a, wn1b, bn1, wn2, bn2)


def kernel(h, x, edge_index, edge_attr, W1e, b1e, W2e, b2e, Wc1, bc1, Wc2,
           Wn1, bn1, Wn2, bn2):
    row = edge_index[0].astype(jnp.int32)
    col = edge_index[1].astype(jnp.int32)
    xpad = jnp.pad(x.astype(jnp.float32), ((0, 0), (0, XW - 3)))
    x0, x1, x2 = x[:, 0], x[:, 1], x[:, 2]

    w1r = W1e[0:H]
    w1c = W1e[H:2 * H]
    w1rad = W1e[2 * H:2 * H + 1]
    w1a = W1e[2 * H + 1:]
    wc2t = Wc2.T
    wn1a = Wn1[0:H]
    wn1b = Wn1[H:2 * H]

    p, q = _prenode(h, w1r, w1c)
    pr, qc, cdrad = _sc_gather(p, q, x0, x1, x2, row, col)
    m_ij, cu = _edge_mlp(pr, qc, cdrad, edge_attr,
                         w1rad, w1a, b1e[None, :], W2e, b2e[None, :],
                         Wc1, bc1[None, :], wc2t)
    zm = jnp.zeros((NPAD, H), jnp.float32)
    zx = jnp.zeros((NPAD, XW), jnp.float32)
    mpart = _sc_scatter_m(m_ij, row, zm)
    xpart = _sc_scatter_x(cu, row, zx)
    h_new, xnew16 = _node_mlp(h, mpart[:N], mpart[NPAD:NPAD + N], xpad,
                              xpart[:N], xpart[NPAD:NPAD + N],
                              wn1a, wn1b, bn1[None, :], Wn2, bn2[None, :])
    return h_new, xnew16[:, :3]


# speedup vs baseline: 1.1293x; 1.1293x over previous
import functools

import jax
import jax.numpy as jnp
from jax import lax
from jax.experimental import pallas as pl
from jax.experimental.pallas import tpu as pltpu
from jax.experimental.pallas import tpu_sc as plsc

N = 10000
H = 128
E = 320000
EDGE_DIM = 16
XW = 16

NC = 2
NS = 16
NW = NC * NS

GC = 80
GEPW = E // NW
GITERS = GEPW // GC


GPAIRS = (GITERS - 1) // 2


@functools.partial(
    pl.kernel,
    out_type=[
        jax.ShapeDtypeStruct((E, H), jnp.float32),
        jax.ShapeDtypeStruct((E, XW), jnp.float32),
    ],
    mesh=plsc.VectorSubcoreMesh(core_axis_name="c", subcore_axis_name="s"),
    compiler_params=pltpu.CompilerParams(needs_layout_passes=False),
    scratch_types=[
        pltpu.VMEM((GEPW,), jnp.int32),
        pltpu.VMEM((GEPW,), jnp.int32),
        pltpu.VMEM((GC, H), jnp.float32),
        pltpu.VMEM((GC, H), jnp.float32),
        pltpu.VMEM((GC, H), jnp.float32),
        pltpu.VMEM((GC, H), jnp.float32),
        pltpu.VMEM((GC, XW), jnp.float32),
        pltpu.VMEM((GC, XW), jnp.float32),
        pltpu.VMEM((N,), jnp.float32),
        pltpu.VMEM((N,), jnp.float32),
        pltpu.VMEM((N,), jnp.float32),
        pltpu.SemaphoreType.DMA,
        pltpu.SemaphoreType.DMA,
    ],
)
def _sc_gather(p_hbm, q_hbm, x0_hbm, x1_hbm, x2_hbm, row_hbm, col_hbm,
               zsum_out, cdrad_out,
               idxr_v, idxc_v, hr0_v, hc0_v, hr1_v, hc1_v, cdrad0_v, cdrad1_v,
               x0_v, x1_v, x2_v, sem0, sem1):
    c = lax.axis_index("c")
    s = lax.axis_index("s")
    base = (s * NC + c) * GEPW
    pltpu.sync_copy(row_hbm.at[pl.ds(base, GEPW)], idxr_v)
    pltpu.sync_copy(col_hbm.at[pl.ds(base, GEPW)], idxc_v)
    pltpu.sync_copy(x0_hbm, x0_v)
    pltpu.sync_copy(x1_hbm, x1_v)
    pltpu.sync_copy(x2_hbm, x2_v)

    def _mk(i, hr_v, hc_v, sem):
        off = i * GC
        c1 = pltpu.make_async_copy(p_hbm.at[idxr_v.at[pl.ds(off, GC)]], hr_v, sem)
        c2 = pltpu.make_async_copy(q_hbm.at[idxc_v.at[pl.ds(off, GC)]], hc_v, sem)
        return c1, c2

    def _start(i, hr_v, hc_v, sem):
        c1, c2 = _mk(i, hr_v, hc_v, sem)
        c1.start()
        c2.start()

    def _wait(i, hr_v, hc_v, sem):
        c1, c2 = _mk(i, hr_v, hc_v, sem)
        c1.wait()
        c2.wait()

    def _geo(i, cdrad_v):
        off = i * GC
        for j in range(GC // 16):
            ir = idxr_v[pl.ds(off + j * 16, 16)]
            ic = idxc_v[pl.ds(off + j * 16, 16)]
            cd0 = plsc.load_gather(x0_v, [ir]) - plsc.load_gather(x0_v, [ic])
            cd1 = plsc.load_gather(x1_v, [ir]) - plsc.load_gather(x1_v, [ic])
            cd2 = plsc.load_gather(x2_v, [ir]) - plsc.load_gather(x2_v, [ic])
            rad = cd0 * cd0 + cd1 * cd1 + cd2 * cd2
            rows = jax.lax.iota(jnp.int32, 16) + j * 16
            for comp, val in ((0, cd0), (1, cd1), (2, cd2), (3, rad)):
                cols = jnp.full((16,), comp, jnp.int32)
                plsc.store_scatter(cdrad_v, [rows, cols], val)

    def _add(d_v, s_v):
        def rbody(r, carry):
            for cc in range(H // 16):
                sl = pl.ds(cc * 16, 16)
                d_v[r, sl] = d_v[r, sl] + s_v[r, sl]
            return carry
        lax.fori_loop(0, GC, rbody, 0)

    def _outs(i, hr_v, hc_v, cdrad_v):
        off = base + i * GC
        _add(hr_v, hc_v)
        pltpu.sync_copy(hr_v, zsum_out.at[pl.ds(off, GC)])
        pltpu.sync_copy(cdrad_v, cdrad_out.at[pl.ds(off, GC)])

    _start(0, hr0_v, hc0_v, sem0)

    def body(p, carry):
        a = 2 * p
        b = a + 1
        _start(b, hr1_v, hc1_v, sem1)
        _geo(a, cdrad0_v)
        _wait(a, hr0_v, hc0_v, sem0)
        _outs(a, hr0_v, hc0_v, cdrad0_v)
        _start(b + 1, hr0_v, hc0_v, sem0)
        _geo(b, cdrad1_v)
        _wait(b, hr1_v, hc1_v, sem1)
        _outs(b, hr1_v, hc1_v, cdrad1_v)
        return carry

    lax.fori_loop(0, GPAIRS, body, 0)
    last = GITERS - 1
    _geo(last, cdrad0_v)
    _wait(last, hr0_v, hc0_v, sem0)
    _outs(last, hr0_v, hc0_v, cdrad0_v)


SC2 = 80
SITERS = E // NW // SC2
SEPC = E // NC
SEPW = E // NW
NPT = 640
NPAD = NPT * NS
RCH = 32


@functools.partial(
    pl.kernel,
    out_type=[
        jax.ShapeDtypeStruct((NC * NPAD, H), jnp.float32),
        jax.ShapeDtypeStruct((NC * NPAD, XW), jnp.float32),
    ],
    mesh=plsc.VectorSubcoreMesh(core_axis_name="c", subcore_axis_name="s"),
    compiler_params=pltpu.CompilerParams(
        needs_layout_passes=False, use_tc_tiling_on_sc=False),
    scratch_types=[
        pltpu.VMEM((SC2,), jnp.int32),
        pltpu.VMEM((SC2,), jnp.int32),
        pltpu.VMEM((SC2, H), jnp.float32),
        pltpu.VMEM((SC2, H), jnp.float32),
        pltpu.VMEM((SC2, XW), jnp.float32),
        pltpu.VMEM((SC2, XW), jnp.float32),
        pltpu.VMEM((RCH, H), jnp.float32),
        pltpu.VMEM((RCH, XW), jnp.float32),
        pltpu.VMEM_SHARED((NPAD, H), jnp.float32),
        pltpu.VMEM_SHARED((NPAD, XW), jnp.float32),
        pltpu.SemaphoreType.DMA,
        pltpu.SemaphoreType.DMA,
    ],
)
def _sc_scatter(m_hbm, cu_hbm, row_hbm, zm_hbm, zx_hbm, mpart, xpart,
                idx0_v, idx1_v, m0_v, m1_v, cu0_v, cu1_v, mrb_v, xrb_v,
                macc, xacc, sem0, sem1):
    c = lax.axis_index("c")
    s = lax.axis_index("s")
    for k in range(NPT // RCH):
        r0 = s * NPT + k * RCH
        pltpu.sync_copy(zm_hbm.at[pl.ds(r0, RCH)], mrb_v)
        pltpu.sync_copy(mrb_v, macc.at[pl.ds(r0, RCH)])
        pltpu.sync_copy(zx_hbm.at[pl.ds(r0, RCH)], xrb_v)
        pltpu.sync_copy(xrb_v, xacc.at[pl.ds(r0, RCH)])
    plsc.subcore_barrier()

    base = c * SEPC + s * SEPW

    def _mk3(i, idx_v, m_v, cu_v, sem):
        off = base + i * SC2
        return (pltpu.make_async_copy(row_hbm.at[pl.ds(off, SC2)], idx_v, sem),
                pltpu.make_async_copy(m_hbm.at[pl.ds(off, SC2)], m_v, sem),
                pltpu.make_async_copy(cu_hbm.at[pl.ds(off, SC2)], cu_v, sem))

    def _start3(i, idx_v, m_v, cu_v, sem):
        for cp in _mk3(i, idx_v, m_v, cu_v, sem):
            cp.start()

    def _wait3(i, idx_v, m_v, cu_v, sem):
        for cp in _mk3(i, idx_v, m_v, cu_v, sem):
            cp.wait()

    def _scat(idx_v, m_v, cu_v):
        pltpu.sync_copy(m_v, macc.at[idx_v], add=True)
        pltpu.sync_copy(cu_v, xacc.at[idx_v], add=True)

    _start3(0, idx0_v, m0_v, cu0_v, sem0)

    def body(p, carry):
        a = 2 * p
        b = a + 1
        _start3(b, idx1_v, m1_v, cu1_v, sem1)
        _wait3(a, idx0_v, m0_v, cu0_v, sem0)
        _scat(idx0_v, m0_v, cu0_v)
        _start3(b + 1, idx0_v, m0_v, cu0_v, sem0)
        _wait3(b, idx1_v, m1_v, cu1_v, sem1)
        _scat(idx1_v, m1_v, cu1_v)
        return carry

    lax.fori_loop(0, (SITERS - 1) // 2, body, 0)
    last = SITERS - 1
    _wait3(last, idx0_v, m0_v, cu0_v, sem0)
    _scat(idx0_v, m0_v, cu0_v)
    plsc.subcore_barrier()
    for k in range(NPT // RCH):
        r0 = s * NPT + k * RCH
        pltpu.sync_copy(macc.at[pl.ds(r0, RCH)], mrb_v)
        pltpu.sync_copy(mrb_v, mpart.at[pl.ds(c * NPAD + r0, RCH)])
        pltpu.sync_copy(xacc.at[pl.ds(r0, RCH)], xrb_v)
        pltpu.sync_copy(xrb_v, xpart.at[pl.ds(c * NPAD + r0, RCH)])


EB = 2000
ENB = E // EB


def _silu(v):
    return v * jax.nn.sigmoid(v)


def _prenode_body(h, w1r, w1c, b1e, p_out, q_out):
    p_out[...] = (jnp.dot(h[...], w1r[...], preferred_element_type=jnp.float32)
                  + b1e[...])
    q_out[...] = jnp.dot(h[...], w1c[...], preferred_element_type=jnp.float32)


def _prenode(h, w1r, w1c, b1e):
    wspec = lambda shp: pl.BlockSpec(shp, lambda i: (0, 0))
    return pl.pallas_call(
        _prenode_body,
        grid=(NNB,),
        in_specs=[pl.BlockSpec((NB, H), lambda i: (i, 0)), wspec((H, H)),
                  wspec((H, H)), wspec((1, H))],
        out_specs=[pl.BlockSpec((NB, H), lambda i: (i, 0)),
                   pl.BlockSpec((NB, H), lambda i: (i, 0))],
        out_shape=[jax.ShapeDtypeStruct((N, H), jnp.float32),
                   jax.ShapeDtypeStruct((N, H), jnp.float32)],
    )(h, w1r, w1c, b1e)


def _edge_body(zsum, cdrad, ea, w1rad, w1a, w2e, b2e,
               wc1, bc1, wc2t, m_out, cu_out):
    lane = lax.broadcasted_iota(jnp.int32, (1, XW), 1)
    cd = jnp.where(lane < 3, cdrad[...], 0.0)
    radial = cdrad[:, 3:4]
    z1 = (zsum[...]
          + jnp.dot(ea[...], w1a[...], preferred_element_type=jnp.float32)
          + radial * w1rad[...])
    t1 = _silu(z1)
    m = _silu(jnp.dot(t1, w2e[...], preferred_element_type=jnp.float32) + b2e[...])
    t3 = _silu(jnp.dot(m, wc1[...], preferred_element_type=jnp.float32) + bc1[...])
    cw = jnp.sum(t3 * wc2t[...], axis=1, keepdims=True)
    m_out[...] = m
    cu_out[...] = cd * cw


def _edge_mlp(zsum, cdrad, ea, w1rad, w1a, w2e, b2e, wc1, bc1, wc2t):
    wspec = lambda shp: pl.BlockSpec(shp, lambda i: (0, 0))
    return pl.pallas_call(
        _edge_body,
        grid=(ENB,),
        in_specs=[
            pl.BlockSpec((EB, H), lambda i: (i, 0)),
            pl.BlockSpec((EB, XW), lambda i: (i, 0)),
            pl.BlockSpec((EB, EDGE_DIM), lambda i: (i, 0)),
            wspec((1, H)), wspec((EDGE_DIM, H)),
            wspec((H, H)), wspec((1, H)),
            wspec((H, H)), wspec((1, H)), wspec((1, H)),
        ],
        out_specs=[
            pl.BlockSpec((EB, H), lambda i: (i, 0)),
            pl.BlockSpec((EB, XW), lambda i: (i, 0)),
        ],
        out_shape=[
            jax.ShapeDtypeStruct((E, H), jnp.float32),
            jax.ShapeDtypeStruct((E, XW), jnp.float32),
        ],
    )(zsum, cdrad, ea, w1rad, w1a, w2e, b2e, wc1, bc1, wc2t)


NB = 1000
NNB = N // NB


def _node_body(h, m0, m1, x16, xp0, xp1, wn1a, wn1b, bn1, wn2, bn2, hn_out, xn_out):
    mi = m0[...] + m1[...]
    z = _silu(jnp.dot(h[...], wn1a[...], preferred_element_type=jnp.float32)
              + jnp.dot(mi, wn1b[...], preferred_element_type=jnp.float32)
              + bn1[...])
    hn_out[...] = h[...] + jnp.dot(z, wn2[...], preferred_element_type=jnp.float32) + bn2[...]
    xn_out[...] = x16[...] + xp0[...] + xp1[...]


def _node_mlp(h, m0, m1, x16, xp0, xp1, wn1a, wn1b, bn1, wn2, bn2):
    wspec = lambda shp: pl.BlockSpec(shp, lambda i: (0, 0))
    return pl.pallas_call(
        _node_body,
        grid=(NNB,),
        in_specs=[
            pl.BlockSpec((NB, H), lambda i: (i, 0)),
            pl.BlockSpec((NB, H), lambda i: (i, 0)),
            pl.BlockSpec((NB, H), lambda i: (i, 0)),
            pl.BlockSpec((NB, XW), lambda i: (i, 0)),
            pl.BlockSpec((NB, XW), lambda i: (i, 0)),
            pl.BlockSpec((NB, XW), lambda i: (i, 0)),
            wspec((H, H)), wspec((H, H)), wspec((1, H)), wspec((H, H)), wspec((1, H)),
        ],
        out_specs=[
            pl.BlockSpec((NB, H), lambda i: (i, 0)),
            pl.BlockSpec((NB, XW), lambda i: (i, 0)),
        ],
        out_shape=[
            jax.ShapeDtypeStruct((N, H), jnp.float32),
            jax.ShapeDtypeStruct((N, XW), jnp.float32),
        ],
    )(h, m0, m1, x16, xp0, xp1, wn1a, wn1b, bn1, wn2, bn2)


def kernel(h, x, edge_index, edge_attr, W1e, b1e, W2e, b2e, Wc1, bc1, Wc2,
           Wn1, bn1, Wn2, bn2):
    row = edge_index[0].astype(jnp.int32)
    col = edge_index[1].astype(jnp.int32)
    xpad = jnp.pad(x.astype(jnp.float32), ((0, 0), (0, XW - 3)))
    x0, x1, x2 = x[:, 0], x[:, 1], x[:, 2]

    w1r = W1e[0:H]
    w1c = W1e[H:2 * H]
    w1rad = W1e[2 * H:2 * H + 1]
    w1a = W1e[2 * H + 1:]
    wc2t = Wc2.T
    wn1a = Wn1[0:H]
    wn1b = Wn1[H:2 * H]

    p, q = _prenode(h, w1r, w1c, b1e[None, :])
    zsum, cdrad = _sc_gather(p, q, x0, x1, x2, row, col)
    m_ij, cu = _edge_mlp(zsum, cdrad, edge_attr,
                         w1rad, w1a, W2e, b2e[None, :],
                         Wc1, bc1[None, :], wc2t)
    zm = jnp.zeros((NPAD, H), jnp.float32)
    zx = jnp.zeros((NPAD, XW), jnp.float32)
    mpart, xpart = _sc_scatter(m_ij, cu, row, zm, zx)
    h_new, xnew16 = _node_mlp(h, mpart[:N], mpart[NPAD:NPAD + N], xpad,
                              xpart[:N], xpart[NPAD:NPAD + N],
                              wn1a, wn1b, bn1[None, :], Wn2, bn2[None, :])
    return h_new, xnew16[:, :3]
